# Initial kernel scaffold; baseline (speedup 1.0000x reference)
import jax
import jax.numpy as jnp
from jax.experimental import pallas as pl


def kernel(x, edge_index, W1_l, W1_r, b1, W2_l, W2_r, b2):
    raise NotImplementedError("write your pallas kernel here")


# speedup vs baseline: 3.9178x; 3.9178x over previous
import functools

import jax
import jax.numpy as jnp
from jax import lax
from jax.experimental import pallas as pl
from jax.experimental.pallas import tpu as pltpu
from jax.experimental.pallas import tpu_sc as plsc

_N = 10000
_E = 320000
_D = 128
_NC = 2
_NS = 16
_NW = _NC * _NS
_EPW = _E // _NW
_CH = 128
_IR = 80
_IRH = _IR // 2
_EPWP = _IR * _CH
_NP = 10112
_RPS = _NP // _NS
_XR = _N + 8


@functools.lru_cache(maxsize=None)
def _sc_agg():
  mesh = plsc.VectorSubcoreMesh(core_axis_name="c", subcore_axis_name="s")

  @functools.partial(
      pl.kernel, mesh=mesh,
      out_type=jax.ShapeDtypeStruct((_NC, _NP, _D), jnp.float32),
      scratch_types=(
          pltpu.VMEM_SHARED((_NP, _D), jnp.float32),
          pltpu.VMEM((_IRH, _CH), jnp.int32),
          pltpu.VMEM((_IRH, _CH), jnp.int32),
          pltpu.VMEM((_CH, _D), jnp.float32),
          pltpu.VMEM((_CH, _D), jnp.float32),
          pltpu.SemaphoreType.DMA,
          pltpu.SemaphoreType.DMA,
      ))
  def sc_agg(x_hbm, src_hbm, dst_hbm, z_hbm, agg_hbm,
             acc_sh, src_v, dst_v, rb0, rb1, sem0, sem1):
    cid = lax.axis_index("c")
    sid = lax.axis_index("s")
    wid = cid * _NS + sid
    r0 = sid * _RPS

    pltpu.sync_copy(z_hbm.at[pl.ds(r0, _RPS)], acc_sh.at[pl.ds(r0, _RPS)])
    plsc.subcore_barrier()

    for p in range(2):
      pltpu.sync_copy(src_hbm.at[wid, pl.ds(p * _IRH, _IRH)], src_v)
      pltpu.sync_copy(dst_hbm.at[wid, pl.ds(p * _IRH, _IRH)], dst_v)

      pltpu.async_copy(x_hbm.at[src_v.at[0]], rb0, sem0)

      @pl.loop(0, _IRH - 2, step=2)
      def _(j):
        pltpu.make_async_copy(x_hbm.at[src_v.at[j]], rb0, sem0).wait()
        pltpu.async_copy(x_hbm.at[src_v.at[j + 1]], rb1, sem1)
        pltpu.sync_copy(rb0, acc_sh.at[dst_v.at[j]], add=True)
        pltpu.make_async_copy(x_hbm.at[src_v.at[j + 1]], rb1, sem1).wait()
        pltpu.async_copy(x_hbm.at[src_v.at[j + 2]], rb0, sem0)
        pltpu.sync_copy(rb1, acc_sh.at[dst_v.at[j + 1]], add=True)

      pltpu.make_async_copy(x_hbm.at[src_v.at[_IRH - 2]], rb0, sem0).wait()
      pltpu.async_copy(x_hbm.at[src_v.at[_IRH - 1]], rb1, sem1)
      pltpu.sync_copy(rb0, acc_sh.at[dst_v.at[_IRH - 2]], add=True)
      pltpu.make_async_copy(x_hbm.at[src_v.at[_IRH - 1]], rb1, sem1).wait()
      pltpu.sync_copy(rb1, acc_sh.at[dst_v.at[_IRH - 1]], add=True)

    plsc.subcore_barrier()
    pltpu.sync_copy(acc_sh.at[pl.ds(r0, _RPS)], agg_hbm.at[cid, pl.ds(r0, _RPS)])

  return sc_agg


@functools.lru_cache(maxsize=None)
def _sc_counts():
  mesh = plsc.VectorSubcoreMesh(core_axis_name="c", subcore_axis_name="s")

  @functools.partial(
      pl.kernel, mesh=mesh,
      out_type=jax.ShapeDtypeStruct((_NC, _NP, _D), jnp.float32),
      scratch_types=(
          pltpu.VMEM_SHARED((_NP, _D), jnp.float32),
          pltpu.VMEM((_IR, _CH), jnp.int32),
          pltpu.VMEM((_CH, _D), jnp.float32),
      ))
  def sc_counts(dst_hbm, zc_hbm, ones_hbm, cnt_hbm, cacc_sh, dst_v, ones_v):
    cid = lax.axis_index("c")
    sid = lax.axis_index("s")
    wid = cid * _NS + sid
    r0 = sid * _RPS

    pltpu.sync_copy(zc_hbm.at[pl.ds(r0, _RPS)], cacc_sh.at[pl.ds(r0, _RPS)])
    pltpu.sync_copy(ones_hbm, ones_v)
    pltpu.sync_copy(dst_hbm.at[wid], dst_v)
    plsc.subcore_barrier()

    @pl.loop(0, _IR)
    def _(j):
      pltpu.sync_copy(ones_v, cacc_sh.at[dst_v.at[j]], add=True)

    plsc.subcore_barrier()
    pltpu.sync_copy(cacc_sh.at[pl.ds(r0, _RPS)], cnt_hbm.at[cid, pl.ds(r0, _RPS)])

  return sc_counts


_BN = 2000


def _dense_body(agg_ref, cnt_ref, x_ref, wl_ref, wr_ref, b_ref, o_ref):
  a = agg_ref[0] + agg_ref[1]
  c = cnt_ref[0] + cnt_ref[1]
  mean = a / jnp.maximum(c[:, 0:1], 1.0)
  o = (jnp.dot(mean, wl_ref[...], preferred_element_type=jnp.float32,
               precision=lax.Precision.HIGHEST)
       + jnp.dot(x_ref[...], wr_ref[...], preferred_element_type=jnp.float32,
                 precision=lax.Precision.HIGHEST)
       + b_ref[...])
  o_ref[...] = jnp.maximum(o, 0.0)


def _dense(agg, cnt, x, wl, wr, b):
  return pl.pallas_call(
      _dense_body,
      grid=(_N // _BN,),
      in_specs=[
          pl.BlockSpec((_NC, _BN, _D), lambda i: (0, i, 0)),
          pl.BlockSpec((_NC, _BN, _D), lambda i: (0, i, 0)),
          pl.BlockSpec((_BN, _D), lambda i: (i, 0)),
          pl.BlockSpec((_D, _D), lambda i: (0, 0)),
          pl.BlockSpec((_D, _D), lambda i: (0, 0)),
          pl.BlockSpec((1, _D), lambda i: (0, 0)),
      ],
      out_specs=pl.BlockSpec((_BN, _D), lambda i: (i, 0)),
      out_shape=jax.ShapeDtypeStruct((_N, _D), jnp.float32),
  )(agg, cnt, x, wl, wr, b)


def kernel(x, edge_index, W1_l, W1_r, b1, W2_l, W2_r, b2):
  npad = _EPWP - _EPW
  src = jnp.concatenate(
      [edge_index[0].reshape(_NW, _EPW),
       jnp.full((_NW, npad), _N, jnp.int32)], axis=1).reshape(_NW, _IR, _CH)
  junk = (_N + (jnp.arange(npad, dtype=jnp.int32) % (_NP - _N)))
  dst = jnp.concatenate(
      [edge_index[1].reshape(_NW, _EPW),
       jnp.broadcast_to(junk, (_NW, npad))], axis=1).reshape(_NW, _IR, _CH)
  zrow = jnp.zeros((_XR - _N, _D), jnp.float32)
  x_aug = jnp.concatenate([x, zrow], axis=0)
  z = jnp.zeros((_NP, _D), jnp.float32)
  ones = jnp.ones((_CH, _D), jnp.float32)

  cnt = _sc_counts()(dst, z, ones)
  agg1 = _sc_agg()(x_aug, src, dst, z)
  h = _dense(agg1, cnt, x, W1_l, W1_r, b1.reshape(1, _D))
  h_aug = jnp.concatenate([h, zrow], axis=0)
  agg2 = _sc_agg()(h_aug, src, dst, z)
  return _dense(agg2, cnt, h, W2_l, W2_r, b2.reshape(1, _D))


# speedup vs baseline: 4.0899x; 1.0439x over previous
import functools

import jax
import jax.numpy as jnp
from jax import lax
from jax.experimental import pallas as pl
from jax.experimental.pallas import tpu as pltpu
from jax.experimental.pallas import tpu_sc as plsc

_N = 10000
_E = 320000
_D = 128
_NC = 2
_NS = 16
_NW = _NC * _NS
_EPW = _E // _NW
_CH = 128
_IR = 80
_IRH = _IR // 2
_EPWP = _IR * _CH
_NP = 10112
_RPS = _NP // _NS
_XR = _N + 8


@functools.lru_cache(maxsize=None)
def _sc_agg():
  mesh = plsc.VectorSubcoreMesh(core_axis_name="c", subcore_axis_name="s")

  @functools.partial(
      pl.kernel, mesh=mesh,
      out_type=jax.ShapeDtypeStruct((_NC, _NP, _D), jnp.float32),
      scratch_types=(
          pltpu.VMEM_SHARED((_NP, _D), jnp.float32),
          pltpu.VMEM((_IRH, _CH), jnp.int32),
          pltpu.VMEM((_IRH, _CH), jnp.int32),
          pltpu.VMEM((_CH, _D), jnp.float32),
          pltpu.VMEM((_CH, _D), jnp.float32),
          pltpu.SemaphoreType.DMA,
          pltpu.SemaphoreType.DMA,
      ))
  def sc_agg(x_hbm, src_hbm, dst_hbm, z_hbm, agg_hbm,
             acc_sh, src_v, dst_v, rb0, rb1, sem0, sem1):
    cid = lax.axis_index("c")
    sid = lax.axis_index("s")
    wid = cid * _NS + sid
    r0 = sid * _RPS

    pltpu.sync_copy(z_hbm.at[pl.ds(r0, _RPS)], acc_sh.at[pl.ds(r0, _RPS)])
    plsc.subcore_barrier()

    for p in range(2):
      pltpu.sync_copy(src_hbm.at[wid, pl.ds(p * _IRH, _IRH)], src_v)
      pltpu.sync_copy(dst_hbm.at[wid, pl.ds(p * _IRH, _IRH)], dst_v)

      pltpu.async_copy(x_hbm.at[src_v.at[0]], rb0, sem0)
      pltpu.async_copy(x_hbm.at[src_v.at[1]], rb1, sem1)

      @pl.loop(0, _IRH - 2, step=2)
      def _(j):
        pltpu.make_async_copy(x_hbm.at[src_v.at[j]], rb0, sem0).wait()
        pltpu.sync_copy(rb0, acc_sh.at[dst_v.at[j]], add=True)
        pltpu.async_copy(x_hbm.at[src_v.at[j + 2]], rb0, sem0)
        pltpu.make_async_copy(x_hbm.at[src_v.at[j + 1]], rb1, sem1).wait()
        pltpu.sync_copy(rb1, acc_sh.at[dst_v.at[j + 1]], add=True)

        @pl.when(j + 3 < _IRH)
        def _():
          pltpu.async_copy(x_hbm.at[src_v.at[j + 3]], rb1, sem1)

      pltpu.make_async_copy(x_hbm.at[src_v.at[_IRH - 2]], rb0, sem0).wait()
      pltpu.sync_copy(rb0, acc_sh.at[dst_v.at[_IRH - 2]], add=True)
      pltpu.make_async_copy(x_hbm.at[src_v.at[_IRH - 1]], rb1, sem1).wait()
      pltpu.sync_copy(rb1, acc_sh.at[dst_v.at[_IRH - 1]], add=True)

    plsc.subcore_barrier()
    pltpu.sync_copy(acc_sh.at[pl.ds(r0, _RPS)], agg_hbm.at[cid, pl.ds(r0, _RPS)])

  return sc_agg


@functools.lru_cache(maxsize=None)
def _sc_counts():
  mesh = plsc.VectorSubcoreMesh(core_axis_name="c", subcore_axis_name="s")

  @functools.partial(
      pl.kernel, mesh=mesh,
      out_type=jax.ShapeDtypeStruct((_NC, _NP, _D), jnp.float32),
      scratch_types=(
          pltpu.VMEM_SHARED((_NP, _D), jnp.float32),
          pltpu.VMEM((_IR, _CH), jnp.int32),
          pltpu.VMEM((_CH, _D), jnp.float32),
      ))
  def sc_counts(dst_hbm, zc_hbm, ones_hbm, cnt_hbm, cacc_sh, dst_v, ones_v):
    cid = lax.axis_index("c")
    sid = lax.axis_index("s")
    wid = cid * _NS + sid
    r0 = sid * _RPS

    pltpu.sync_copy(zc_hbm.at[pl.ds(r0, _RPS)], cacc_sh.at[pl.ds(r0, _RPS)])
    pltpu.sync_copy(ones_hbm, ones_v)
    pltpu.sync_copy(dst_hbm.at[wid], dst_v)
    plsc.subcore_barrier()

    @pl.loop(0, _IR)
    def _(j):
      pltpu.sync_copy(ones_v, cacc_sh.at[dst_v.at[j]], add=True)

    plsc.subcore_barrier()
    pltpu.sync_copy(cacc_sh.at[pl.ds(r0, _RPS)], cnt_hbm.at[cid, pl.ds(r0, _RPS)])

  return sc_counts


_BN = 2000


def _dense_body(agg_ref, cnt_ref, x_ref, wl_ref, wr_ref, b_ref, o_ref):
  a = agg_ref[0] + agg_ref[1]
  c = cnt_ref[0] + cnt_ref[1]
  mean = a / jnp.maximum(c[:, 0:1], 1.0)
  o = (jnp.dot(mean, wl_ref[...], preferred_element_type=jnp.float32,
               precision=lax.Precision.HIGHEST)
       + jnp.dot(x_ref[...], wr_ref[...], preferred_element_type=jnp.float32,
                 precision=lax.Precision.HIGHEST)
       + b_ref[...])
  o_ref[...] = jnp.maximum(o, 0.0)


def _dense(agg, cnt, x, wl, wr, b):
  return pl.pallas_call(
      _dense_body,
      grid=(_N // _BN,),
      in_specs=[
          pl.BlockSpec((_NC, _BN, _D), lambda i: (0, i, 0)),
          pl.BlockSpec((_NC, _BN, _D), lambda i: (0, i, 0)),
          pl.BlockSpec((_BN, _D), lambda i: (i, 0)),
          pl.BlockSpec((_D, _D), lambda i: (0, 0)),
          pl.BlockSpec((_D, _D), lambda i: (0, 0)),
          pl.BlockSpec((1, _D), lambda i: (0, 0)),
      ],
      out_specs=pl.BlockSpec((_BN, _D), lambda i: (i, 0)),
      out_shape=jax.ShapeDtypeStruct((_N, _D), jnp.float32),
  )(agg, cnt, x, wl, wr, b)


def kernel(x, edge_index, W1_l, W1_r, b1, W2_l, W2_r, b2):
  npad = _EPWP - _EPW
  src = jnp.concatenate(
      [edge_index[0].reshape(_NW, _EPW),
       jnp.full((_NW, npad), _N, jnp.int32)], axis=1).reshape(_NW, _IR, _CH)
  junk = (_N + (jnp.arange(npad, dtype=jnp.int32) % (_NP - _N)))
  dst = jnp.concatenate(
      [edge_index[1].reshape(_NW, _EPW),
       jnp.broadcast_to(junk, (_NW, npad))], axis=1).reshape(_NW, _IR, _CH)
  zrow = jnp.zeros((_XR - _N, _D), jnp.float32)
  x_aug = jnp.concatenate([x, zrow], axis=0)
  z = jnp.zeros((_NP, _D), jnp.float32)
  ones = jnp.ones((_CH, _D), jnp.float32)

  cnt = _sc_counts()(dst, z, ones)
  agg1 = _sc_agg()(x_aug, src, dst, z)
  h = _dense(agg1, cnt, x, W1_l, W1_r, b1.reshape(1, _D))
  h_aug = jnp.concatenate([h, zrow], axis=0)
  agg2 = _sc_agg()(h_aug, src, dst, z)
  return _dense(agg2, cnt, h, W2_l, W2_r, b2.reshape(1, _D))


# speedup vs baseline: 4.0920x; 1.0005x over previous
import functools

import jax
import jax.numpy as jnp
from jax import lax
from jax.experimental import pallas as pl
from jax.experimental.pallas import tpu as pltpu
from jax.experimental.pallas import tpu_sc as plsc

_N = 10000
_E = 320000
_D = 128
_NC = 2
_NS = 16
_NW = _NC * _NS
_EPW = _E // _NW
_CH = 128
_IR = 80
_IRH = _IR // 2
_EPWP = _IR * _CH
_NP = 10112
_RPS = _NP // _NS
_XR = _N + 8


@functools.lru_cache(maxsize=None)
def _sc_agg():
  mesh = plsc.VectorSubcoreMesh(core_axis_name="c", subcore_axis_name="s")

  @functools.partial(
      pl.kernel, mesh=mesh,
      out_type=jax.ShapeDtypeStruct((_NC, _NP, _D), jnp.float32),
      scratch_types=(
          pltpu.VMEM_SHARED((_NP, _D), jnp.float32),
          pltpu.VMEM((_IRH, _CH), jnp.int32),
          pltpu.VMEM((_IRH, _CH), jnp.int32),
          pltpu.VMEM((_CH, _D), jnp.float32),
          pltpu.VMEM((_CH, _D), jnp.float32),
          pltpu.SemaphoreType.DMA,
          pltpu.SemaphoreType.DMA,
      ))
  def sc_agg(x_hbm, src_hbm, dst_hbm, z_hbm, agg_hbm,
             acc_sh, src_v, dst_v, rb0, rb1, sem0, sem1):
    cid = lax.axis_index("c")
    sid = lax.axis_index("s")
    wid = cid * _NS + sid
    r0 = sid * _RPS

    pltpu.sync_copy(z_hbm.at[pl.ds(r0, _RPS)], acc_sh.at[pl.ds(r0, _RPS)])
    plsc.subcore_barrier()

    for p in range(2):
      pltpu.sync_copy(src_hbm.at[wid, pl.ds(p * _IRH, _IRH)], src_v)
      pltpu.sync_copy(dst_hbm.at[wid, pl.ds(p * _IRH, _IRH)], dst_v)

      pltpu.async_copy(x_hbm.at[src_v.at[0]], rb0, sem0)
      pltpu.async_copy(x_hbm.at[src_v.at[1]], rb1, sem1)

      @pl.loop(0, _IRH - 2, step=2)
      def _(j):
        pltpu.make_async_copy(x_hbm.at[src_v.at[j]], rb0, sem0).wait()
        pltpu.sync_copy(rb0, acc_sh.at[dst_v.at[j]], add=True)
        pltpu.async_copy(x_hbm.at[src_v.at[j + 2]], rb0, sem0)
        pltpu.make_async_copy(x_hbm.at[src_v.at[j + 1]], rb1, sem1).wait()
        pltpu.sync_copy(rb1, acc_sh.at[dst_v.at[j + 1]], add=True)

        @pl.when(j + 3 < _IRH)
        def _():
          pltpu.async_copy(x_hbm.at[src_v.at[j + 3]], rb1, sem1)

      pltpu.make_async_copy(x_hbm.at[src_v.at[_IRH - 2]], rb0, sem0).wait()
      pltpu.sync_copy(rb0, acc_sh.at[dst_v.at[_IRH - 2]], add=True)
      pltpu.make_async_copy(x_hbm.at[src_v.at[_IRH - 1]], rb1, sem1).wait()
      pltpu.sync_copy(rb1, acc_sh.at[dst_v.at[_IRH - 1]], add=True)

    plsc.subcore_barrier()
    pltpu.sync_copy(acc_sh.at[pl.ds(r0, _RPS)], agg_hbm.at[cid, pl.ds(r0, _RPS)])

  return sc_agg


@functools.lru_cache(maxsize=None)
def _sc_counts():
  mesh = plsc.VectorSubcoreMesh(core_axis_name="c", subcore_axis_name="s")

  @functools.partial(
      pl.kernel, mesh=mesh,
      out_type=jax.ShapeDtypeStruct((_NC, _NP, _D), jnp.float32),
      scratch_types=(
          pltpu.VMEM_SHARED((_NP, _D), jnp.float32),
          pltpu.VMEM((_IR, _CH), jnp.int32),
          pltpu.VMEM((_CH, _D), jnp.float32),
          pltpu.SemaphoreType.DMA,
      ))
  def sc_counts(dst_hbm, zc_hbm, ones_hbm, cnt_hbm, cacc_sh, dst_v, ones_v,
                sem):
    cid = lax.axis_index("c")
    sid = lax.axis_index("s")
    wid = cid * _NS + sid
    r0 = sid * _RPS

    pltpu.sync_copy(zc_hbm.at[pl.ds(r0, _RPS)], cacc_sh.at[pl.ds(r0, _RPS)])
    pltpu.sync_copy(ones_hbm, ones_v)
    pltpu.sync_copy(dst_hbm.at[wid], dst_v)
    plsc.subcore_barrier()

    @pl.loop(0, _IR, step=16)
    def _(j0):
      @pl.loop(0, 16)
      def _(i):
        pltpu.async_copy(ones_v, cacc_sh.at[dst_v.at[j0 + i]], sem, add=True)

      @pl.loop(0, 16)
      def _(i):
        pltpu.make_async_copy(ones_v, cacc_sh.at[dst_v.at[j0]], sem).wait()

    plsc.subcore_barrier()
    pltpu.sync_copy(cacc_sh.at[pl.ds(r0, _RPS)], cnt_hbm.at[cid, pl.ds(r0, _RPS)])

  return sc_counts


_BN = 2000


def _dense_body(agg_ref, cnt_ref, x_ref, wl_ref, wr_ref, b_ref, o_ref):
  a = agg_ref[0] + agg_ref[1]
  c = cnt_ref[0] + cnt_ref[1]
  mean = a / jnp.maximum(c[:, 0:1], 1.0)
  o = (jnp.dot(mean, wl_ref[...], preferred_element_type=jnp.float32,
               precision=lax.Precision.HIGHEST)
       + jnp.dot(x_ref[...], wr_ref[...], preferred_element_type=jnp.float32,
                 precision=lax.Precision.HIGHEST)
       + b_ref[...])
  o_ref[...] = jnp.maximum(o, 0.0)


def _dense(agg, cnt, x, wl, wr, b):
  return pl.pallas_call(
      _dense_body,
      grid=(_N // _BN,),
      in_specs=[
          pl.BlockSpec((_NC, _BN, _D), lambda i: (0, i, 0)),
          pl.BlockSpec((_NC, _BN, _D), lambda i: (0, i, 0)),
          pl.BlockSpec((_BN, _D), lambda i: (i, 0)),
          pl.BlockSpec((_D, _D), lambda i: (0, 0)),
          pl.BlockSpec((_D, _D), lambda i: (0, 0)),
          pl.BlockSpec((1, _D), lambda i: (0, 0)),
      ],
      out_specs=pl.BlockSpec((_BN, _D), lambda i: (i, 0)),
      out_shape=jax.ShapeDtypeStruct((_N, _D), jnp.float32),
  )(agg, cnt, x, wl, wr, b)


def kernel(x, edge_index, W1_l, W1_r, b1, W2_l, W2_r, b2):
  npad = _EPWP - _EPW
  src = jnp.concatenate(
      [edge_index[0].reshape(_NW, _EPW),
       jnp.full((_NW, npad), _N, jnp.int32)], axis=1).reshape(_NW, _IR, _CH)
  junk = (_N + (jnp.arange(npad, dtype=jnp.int32) % (_NP - _N)))
  dst = jnp.concatenate(
      [edge_index[1].reshape(_NW, _EPW),
       jnp.broadcast_to(junk, (_NW, npad))], axis=1).reshape(_NW, _IR, _CH)
  zrow = jnp.zeros((_XR - _N, _D), jnp.float32)
  x_aug = jnp.concatenate([x, zrow], axis=0)
  z = jnp.zeros((_NP, _D), jnp.float32)
  ones = jnp.ones((_CH, _D), jnp.float32)

  cnt = _sc_counts()(dst, z, ones)
  agg1 = _sc_agg()(x_aug, src, dst, z)
  h = _dense(agg1, cnt, x, W1_l, W1_r, b1.reshape(1, _D))
  h_aug = jnp.concatenate([h, zrow], axis=0)
  agg2 = _sc_agg()(h_aug, src, dst, z)
  return _dense(agg2, cnt, h, W2_l, W2_r, b2.reshape(1, _D))
